# baseline (speedup 1.0000x reference)
import functools

import jax
import jax.numpy as jnp
from jax import lax
from jax.experimental import pallas as pl
from jax.experimental.pallas import tpu as pltpu
from jax.experimental.pallas import tpu_sc as plsc

_MAX_NEIGHBORS = 32
_CUTOFF = 10.0
_TOL = 0.15
_N = 1024
_R = 128
_NBLK = _N // _R
_NSHIFT = 27
_K1 = _MAX_NEIGHBORS + 1
_OUTW = 64


def _select_body(q2_ref, k2_ref, pos_ref, kt_ref, sd_ref, gi_ref, mask_ref,
                 dmin2, gidx):
    i = pl.program_id(0)
    s = pl.program_id(1)
    p = pos_ref[0]
    kt = kt_ref[0]
    m = jax.lax.dot_general(
        p, kt, dimension_numbers=(((1,), (0,)), ((), ())),
        preferred_element_type=jnp.float32,
    )
    q2 = q2_ref[0]
    k2 = k2_ref[0]
    d2 = (q2 + k2) - 2.0 * m
    colj = jax.lax.broadcasted_iota(jnp.int32, (_R, _N), 1)

    @pl.when(s == 0)
    def _():
        dmin2[...] = d2
        gidx[...] = colj

    @pl.when((s > 0) & (s != 13))
    def _():
        dm = dmin2[...]
        upd = d2 < dm
        dmin2[...] = jnp.where(upd, d2, dm)
        gidx[...] = jnp.where(upd, s * _N + colj, gidx[...])

    @pl.when(s == 13)
    def _():
        rowg = i * _R + jax.lax.broadcasted_iota(jnp.int32, (_R, _N), 0)
        dm = dmin2[...]
        upd = (d2 < dm) & (colj != rowg)
        dmin2[...] = jnp.where(upd, d2, dm)
        gidx[...] = jnp.where(upd, s * _N + colj, gidx[...])

    @pl.when(s == _NSHIFT - 1)
    def _():
        inf = jnp.float32(jnp.inf)
        dcur = jnp.sqrt(jnp.maximum(dmin2[...], 1e-12))
        dcur = jnp.where(dcur > _CUTOFF, inf, dcur)
        g = gidx[...]
        big_i = jnp.int32(2**30)
        c115 = jnp.float32(1.0 + _TOL)
        csum = jnp.zeros((_R, 1), jnp.float32)
        found = jnp.zeros((_R, 1), jnp.bool_)
        m_sel = jnp.full((_R, 1), _MAX_NEIGHBORS, jnp.int32)
        r_sel = jnp.full((_R, 1), _CUTOFF, jnp.float32)
        sd_cols = []
        gi_cols = []
        for t in range(_K1):
            v = jnp.min(dcur, axis=1, keepdims=True)
            tie = dcur == v
            gm = jnp.min(jnp.where(tie, g, big_i), axis=1, keepdims=True)
            if t < _MAX_NEIGHBORS:
                sd_cols.append(v)
                gi_cols.append(gm)
                dcur = jnp.where(g == gm, inf, dcur)
            if t >= 3:
                r_m = csum / jnp.float32(t - 2)
                ok = r_m < v
                newly = ok & (~found)
                m_sel = jnp.where(newly, t, m_sel)
                r_sel = jnp.where(newly, r_m, r_sel)
                found = found | ok
            csum = csum + v
        dist_cols = []
        mask_cols = []
        for t in range(_MAX_NEIGHBORS):
            v = sd_cols[t]
            mk = (t < m_sel) & (v <= r_sel * c115) & (v < inf)
            mask_cols.append(jnp.where(mk, jnp.int32(1), jnp.int32(0)))
            dist_cols.append(jnp.where(mk, v, 0.0))
        sd_ref[0] = jnp.concatenate(dist_cols, axis=1)
        gi_ref[0] = jnp.concatenate(gi_cols, axis=1)
        mask_ref[0] = jnp.concatenate(mask_cols, axis=1)


def _topk_candidates(q2, k2, pos, keys_t):
    q2_3 = q2.reshape(_NBLK, _R, 1)
    k2_3 = k2.reshape(1, _NSHIFT, _N).transpose(1, 0, 2)
    pos_3 = pos.reshape(_NBLK, _R, 3)
    kt_3 = keys_t.reshape(3, _NSHIFT, _N).transpose(1, 0, 2)
    dist, gi, mask = pl.pallas_call(
        _select_body,
        grid=(_NBLK, _NSHIFT),
        in_specs=[
            pl.BlockSpec((1, _R, 1), lambda i, s: (i, 0, 0)),
            pl.BlockSpec((1, 1, _N), lambda i, s: (s, 0, 0)),
            pl.BlockSpec((1, _R, 3), lambda i, s: (i, 0, 0)),
            pl.BlockSpec((1, 3, _N), lambda i, s: (s, 0, 0)),
        ],
        out_specs=[
            pl.BlockSpec((1, _R, _MAX_NEIGHBORS), lambda i, s: (i, 0, 0)),
            pl.BlockSpec((1, _R, _MAX_NEIGHBORS), lambda i, s: (i, 0, 0)),
            pl.BlockSpec((1, _R, _MAX_NEIGHBORS), lambda i, s: (i, 0, 0)),
        ],
        out_shape=[
            jax.ShapeDtypeStruct((_NBLK, _R, _MAX_NEIGHBORS), jnp.float32),
            jax.ShapeDtypeStruct((_NBLK, _R, _MAX_NEIGHBORS), jnp.int32),
            jax.ShapeDtypeStruct((_NBLK, _R, _MAX_NEIGHBORS), jnp.int32),
        ],
        scratch_shapes=[
            pltpu.VMEM((_R, _N), jnp.float32),
            pltpu.VMEM((_R, _N), jnp.int32),
        ],
    )(q2_3, k2_3, pos_3, kt_3)
    dist = dist.reshape(_N, _MAX_NEIGHBORS)
    gi = gi.reshape(_N, _MAX_NEIGHBORS)
    mask = mask.reshape(_N, _MAX_NEIGHBORS)
    return dist, gi, mask


_NEDGE = _N * _MAX_NEIGHBORS
_KPAD = 8
_NW = 32
_EPW = _NEDGE // _NW
_L = 16


def _sc_vec_body(keys_hbm, gi_hbm, mask_hbm, pos_hbm, out_hbm,
                 keys_v, idx_v, mask_v, pos_v, out_v):
    wid = lax.axis_index("s") * 2 + lax.axis_index("c")
    pltpu.sync_copy(keys_hbm, keys_v)
    pltpu.sync_copy(gi_hbm.at[wid], idx_v)
    pltpu.sync_copy(mask_hbm.at[wid], mask_v)
    pltpu.sync_copy(pos_hbm, pos_v)

    lane = lax.iota(jnp.int32, _L)
    zero = jnp.zeros((_L,), jnp.float32)
    base = wid * _EPW
    for t in range(_EPW // _L):
        e = t * _L + lane
        g3 = idx_v[pl.ds(t * _L, _L)] * 3
        i3 = (lax.shift_right_logical(base + e, 5)) * 3
        keep = mask_v[pl.ds(t * _L, _L)] != 0
        for c in range(3):
            kc = plsc.load_gather(keys_v, [g3 + c])
            pc = plsc.load_gather(pos_v, [i3 + c])
            vc = jnp.where(keep, kc - pc, zero)
            plsc.store_scatter(out_v, [e * 3 + c], vc)
    pltpu.sync_copy(out_v, out_hbm.at[wid])


def _sc_vec(keys_flat, gi, mask_i32, pos):
    gi_2 = gi.reshape(_NW, _EPW)
    mask_2 = mask_i32.reshape(_NW, _EPW)
    pos_flat = pos.reshape(_N * 3)
    mesh = plsc.VectorSubcoreMesh(core_axis_name="c", subcore_axis_name="s")
    fn = functools.partial(
        pl.kernel,
        mesh=mesh,
        compiler_params=pltpu.CompilerParams(needs_layout_passes=False),
        out_type=jax.ShapeDtypeStruct((_NW, _EPW * 3), jnp.float32),
        scratch_types=[
            pltpu.VMEM((_NSHIFT * _N * 3,), jnp.float32),
            pltpu.VMEM((_EPW,), jnp.int32),
            pltpu.VMEM((_EPW,), jnp.int32),
            pltpu.VMEM((_N * 3,), jnp.float32),
            pltpu.VMEM((_EPW * 3,), jnp.float32),
        ],
    )(_sc_vec_body)
    out = fn(keys_flat, gi_2, mask_2, pos_flat)
    return out.reshape(_N, _MAX_NEIGHBORS, 3)


def kernel(frac_coords, cell):
    n = frac_coords.shape[0]
    pos = frac_coords @ cell
    r = jnp.arange(-1, 2)
    shifts = jnp.stack(jnp.meshgrid(r, r, r, indexing="ij"), axis=-1)
    shifts = shifts.reshape(-1, 3).astype(cell.dtype)
    offsets = shifts @ cell
    keys = (pos[None, :, :] + offsets[:, None, :]).reshape(-1, 3)
    q2 = jnp.sum(pos * pos, axis=1)
    k2 = jnp.sum(keys * keys, axis=1)

    dist, idx_k, mask_i32 = _topk_candidates(q2, k2, pos, keys.T)
    mask = mask_i32 != 0

    vec = _sc_vec(keys.reshape(-1), idx_k, mask_i32, pos)
    dst = idx_k % n
    src = jnp.broadcast_to(jnp.arange(n)[:, None], dst.shape)
    edge_index = jnp.stack([src.reshape(-1), dst.reshape(-1)], axis=0)
    return edge_index, vec, dist, mask


# speedup vs baseline: 1.1348x; 1.1348x over previous
import functools

import jax
import jax.numpy as jnp
from jax import lax
from jax.experimental import pallas as pl
from jax.experimental.pallas import tpu as pltpu
from jax.experimental.pallas import tpu_sc as plsc

_MAX_NEIGHBORS = 32
_CUTOFF = 10.0
_TOL = 0.15
_N = 1024
_R = 128
_NBLK = _N // _R
_NSHIFT = 27
_K1 = _MAX_NEIGHBORS + 1
_OUTW = 64


def _select_body(q2_ref, k2_ref, pos_ref, kt_ref, sd_ref, gi_ref, mask_ref,
                 dmin2, gidx):
    i = pl.program_id(0)
    s = pl.program_id(1)
    p = pos_ref[0]
    kt = kt_ref[0]
    m = jax.lax.dot_general(
        p, kt, dimension_numbers=(((1,), (0,)), ((), ())),
        preferred_element_type=jnp.float32,
    )
    q2 = q2_ref[0]
    k2 = k2_ref[0]
    d2 = (q2 + k2) - 2.0 * m
    colj = jax.lax.broadcasted_iota(jnp.int32, (_R, _N), 1)

    @pl.when(s == 0)
    def _():
        dmin2[...] = d2
        gidx[...] = colj

    @pl.when((s > 0) & (s != 13))
    def _():
        dm = dmin2[...]
        upd = d2 < dm
        dmin2[...] = jnp.where(upd, d2, dm)
        gidx[...] = jnp.where(upd, s * _N + colj, gidx[...])

    @pl.when(s == 13)
    def _():
        rowg = i * _R + jax.lax.broadcasted_iota(jnp.int32, (_R, _N), 0)
        dm = dmin2[...]
        upd = (d2 < dm) & (colj != rowg)
        dmin2[...] = jnp.where(upd, d2, dm)
        gidx[...] = jnp.where(upd, s * _N + colj, gidx[...])

    @pl.when(s == _NSHIFT - 1)
    def _():
        inf = jnp.float32(jnp.inf)
        dcur = jnp.sqrt(jnp.maximum(dmin2[...], 1e-12))
        dcur = jnp.where(dcur > _CUTOFF, inf, dcur)
        g = gidx[...]
        big_i = jnp.int32(2**30)
        c115 = jnp.float32(1.0 + _TOL)
        csum = jnp.zeros((_R, 1), jnp.float32)
        found = jnp.zeros((_R, 1), jnp.bool_)
        m_sel = jnp.full((_R, 1), _MAX_NEIGHBORS, jnp.int32)
        r_sel = jnp.full((_R, 1), _CUTOFF, jnp.float32)
        sd_cols = []
        gi_cols = []
        for t in range(_K1):
            v = jnp.min(dcur, axis=1, keepdims=True)
            tie = dcur == v
            gm = jnp.min(jnp.where(tie, g, big_i), axis=1, keepdims=True)
            if t < _MAX_NEIGHBORS:
                sd_cols.append(v)
                gi_cols.append(gm)
                dcur = jnp.where(g == gm, inf, dcur)
            if t >= 3:
                r_m = csum / jnp.float32(t - 2)
                ok = r_m < v
                newly = ok & (~found)
                m_sel = jnp.where(newly, t, m_sel)
                r_sel = jnp.where(newly, r_m, r_sel)
                found = found | ok
            csum = csum + v
        dist_cols = []
        mask_cols = []
        for t in range(_MAX_NEIGHBORS):
            v = sd_cols[t]
            mk = (t < m_sel) & (v <= r_sel * c115) & (v < inf)
            mask_cols.append(jnp.where(mk, jnp.int32(1), jnp.int32(0)))
            dist_cols.append(jnp.where(mk, v, 0.0))
        sd_ref[0] = jnp.concatenate(dist_cols, axis=1)
        gi_ref[0] = jnp.concatenate(gi_cols, axis=1)
        mask_ref[0] = jnp.concatenate(mask_cols, axis=1)


def _topk_candidates(q2, k2, pos, keys_t):
    q2_3 = q2.reshape(_NBLK, _R, 1)
    k2_3 = k2.reshape(1, _NSHIFT, _N).transpose(1, 0, 2)
    pos_3 = pos.reshape(_NBLK, _R, 3)
    kt_3 = keys_t.reshape(3, _NSHIFT, _N).transpose(1, 0, 2)
    dist, gi, mask = pl.pallas_call(
        _select_body,
        grid=(_NBLK, _NSHIFT),
        in_specs=[
            pl.BlockSpec((1, _R, 1), lambda i, s: (i, 0, 0)),
            pl.BlockSpec((1, 1, _N), lambda i, s: (s, 0, 0)),
            pl.BlockSpec((1, _R, 3), lambda i, s: (i, 0, 0)),
            pl.BlockSpec((1, 3, _N), lambda i, s: (s, 0, 0)),
        ],
        out_specs=[
            pl.BlockSpec((1, _R, _MAX_NEIGHBORS), lambda i, s: (i, 0, 0)),
            pl.BlockSpec((1, _R, _MAX_NEIGHBORS), lambda i, s: (i, 0, 0)),
            pl.BlockSpec((1, _R, _MAX_NEIGHBORS), lambda i, s: (i, 0, 0)),
        ],
        out_shape=[
            jax.ShapeDtypeStruct((_NBLK, _R, _MAX_NEIGHBORS), jnp.float32),
            jax.ShapeDtypeStruct((_NBLK, _R, _MAX_NEIGHBORS), jnp.int32),
            jax.ShapeDtypeStruct((_NBLK, _R, _MAX_NEIGHBORS), jnp.int32),
        ],
        scratch_shapes=[
            pltpu.VMEM((_R, _N), jnp.float32),
            pltpu.VMEM((_R, _N), jnp.int32),
        ],
    )(q2_3, k2_3, pos_3, kt_3)
    dist = dist.reshape(_N, _MAX_NEIGHBORS)
    gi = gi.reshape(_N, _MAX_NEIGHBORS)
    mask = mask.reshape(_N, _MAX_NEIGHBORS)
    return dist, gi, mask


_NEDGE = _N * _MAX_NEIGHBORS
_KPAD = 8
_NW = 32
_EPW = _NEDGE // _NW
_L = 16


def _sc_vec_body(gi_hbm, mask_hbm, pos_hbm, off_hbm, out_hbm,
                 idx_v, mask_v, pos_v, off_v, out_v):
    wid = lax.axis_index("s") * 2 + lax.axis_index("c")
    pltpu.sync_copy(gi_hbm.at[wid], idx_v)
    pltpu.sync_copy(mask_hbm.at[wid], mask_v)
    pltpu.sync_copy(pos_hbm, pos_v)
    pltpu.sync_copy(off_hbm, off_v)

    lane = lax.iota(jnp.int32, _L)
    zero = jnp.zeros((_L,), jnp.float32)
    base = wid * _EPW
    for t in range(_EPW // _L):
        e = t * _L + lane
        g = idx_v[pl.ds(t * _L, _L)]
        j3 = jnp.bitwise_and(g, _N - 1) * 3
        s3 = lax.shift_right_logical(g, 10) * 3
        i3 = (lax.shift_right_logical(base + e, 5)) * 3
        keep = mask_v[pl.ds(t * _L, _L)] != 0
        for c in range(3):
            kc = plsc.load_gather(pos_v, [j3 + c]) + plsc.load_gather(off_v, [s3 + c])
            pc = plsc.load_gather(pos_v, [i3 + c])
            vc = jnp.where(keep, kc - pc, zero)
            plsc.store_scatter(out_v, [e * 3 + c], vc)
    pltpu.sync_copy(out_v, out_hbm.at[wid])


def _sc_vec(gi, mask_i32, pos, offsets):
    gi_2 = gi.reshape(_NW, _EPW)
    mask_2 = mask_i32.reshape(_NW, _EPW)
    pos_flat = pos.reshape(_N * 3)
    off_flat = offsets.reshape(_NSHIFT * 3)
    mesh = plsc.VectorSubcoreMesh(core_axis_name="c", subcore_axis_name="s")
    fn = functools.partial(
        pl.kernel,
        mesh=mesh,
        compiler_params=pltpu.CompilerParams(needs_layout_passes=False),
        out_type=jax.ShapeDtypeStruct((_NW, _EPW * 3), jnp.float32),
        scratch_types=[
            pltpu.VMEM((_EPW,), jnp.int32),
            pltpu.VMEM((_EPW,), jnp.int32),
            pltpu.VMEM((_N * 3,), jnp.float32),
            pltpu.VMEM((_NSHIFT * 3,), jnp.float32),
            pltpu.VMEM((_EPW * 3,), jnp.float32),
        ],
    )(_sc_vec_body)
    out = fn(gi_2, mask_2, pos_flat, off_flat)
    return out.reshape(_N, _MAX_NEIGHBORS, 3)


def kernel(frac_coords, cell):
    n = frac_coords.shape[0]
    pos = frac_coords @ cell
    r = jnp.arange(-1, 2)
    shifts = jnp.stack(jnp.meshgrid(r, r, r, indexing="ij"), axis=-1)
    shifts = shifts.reshape(-1, 3).astype(cell.dtype)
    offsets = shifts @ cell
    keys = (pos[None, :, :] + offsets[:, None, :]).reshape(-1, 3)
    pos, keys = jax.lax.optimization_barrier((pos, keys))
    q2 = jnp.sum(pos * pos, axis=1)
    k2 = jnp.sum(keys * keys, axis=1)

    dist, idx_k, mask_i32 = _topk_candidates(q2, k2, pos, keys.T)
    mask = mask_i32 != 0

    vec = _sc_vec(idx_k, mask_i32, pos, offsets)
    dst = idx_k % n
    src = jnp.broadcast_to(jnp.arange(n)[:, None], dst.shape)
    edge_index = jnp.stack([src.reshape(-1), dst.reshape(-1)], axis=0)
    return edge_index, vec, dist, mask


# speedup vs baseline: 1.5392x; 1.3563x over previous
import functools

import jax
import jax.numpy as jnp
from jax import lax
from jax.experimental import pallas as pl
from jax.experimental.pallas import tpu as pltpu
from jax.experimental.pallas import tpu_sc as plsc

_MAX_NEIGHBORS = 32
_CUTOFF = 10.0
_TOL = 0.15
_N = 1024
_R = 256
_NBLK = _N // _R
_NSHIFT = 27
_K1 = _MAX_NEIGHBORS + 1
_OUTW = 64


def _select_body(q2_ref, k2_ref, pos_ref, kt_ref, sd_ref, gi_ref, mask_ref,
                 dmin2, gidx):
    i = pl.program_id(0)
    s = pl.program_id(1)
    p = pos_ref[0]
    kt = kt_ref[0]
    m = jax.lax.dot_general(
        p, kt, dimension_numbers=(((1,), (0,)), ((), ())),
        preferred_element_type=jnp.float32,
    )
    q2 = q2_ref[0]
    k2 = k2_ref[0]
    d2 = (q2 + k2) - 2.0 * m
    colj = jax.lax.broadcasted_iota(jnp.int32, (_R, _N), 1)

    @pl.when(s == 0)
    def _():
        dmin2[...] = d2
        gidx[...] = colj

    @pl.when((s > 0) & (s != 13))
    def _():
        dm = dmin2[...]
        upd = d2 < dm
        dmin2[...] = jnp.where(upd, d2, dm)
        gidx[...] = jnp.where(upd, s * _N + colj, gidx[...])

    @pl.when(s == 13)
    def _():
        rowg = i * _R + jax.lax.broadcasted_iota(jnp.int32, (_R, _N), 0)
        dm = dmin2[...]
        upd = (d2 < dm) & (colj != rowg)
        dmin2[...] = jnp.where(upd, d2, dm)
        gidx[...] = jnp.where(upd, s * _N + colj, gidx[...])

    @pl.when(s == _NSHIFT - 1)
    def _():
        inf = jnp.float32(jnp.inf)
        dcur = jnp.sqrt(jnp.maximum(dmin2[...], 1e-12))
        dcur = jnp.where(dcur > _CUTOFF, inf, dcur)
        g = gidx[...]
        big_i = jnp.int32(2**30)
        c115 = jnp.float32(1.0 + _TOL)
        csum = jnp.zeros((_R, 1), jnp.float32)
        found = jnp.zeros((_R, 1), jnp.bool_)
        m_sel = jnp.full((_R, 1), _MAX_NEIGHBORS, jnp.int32)
        r_sel = jnp.full((_R, 1), _CUTOFF, jnp.float32)
        sd_cols = []
        gi_cols = []
        for t in range(_K1):
            v = jnp.min(dcur, axis=1, keepdims=True)
            tie = dcur == v
            gm = jnp.min(jnp.where(tie, g, big_i), axis=1, keepdims=True)
            if t < _MAX_NEIGHBORS:
                sd_cols.append(v)
                gi_cols.append(gm)
                dcur = jnp.where(g == gm, inf, dcur)
            if t >= 3:
                r_m = csum / jnp.float32(t - 2)
                ok = r_m < v
                newly = ok & (~found)
                m_sel = jnp.where(newly, t, m_sel)
                r_sel = jnp.where(newly, r_m, r_sel)
                found = found | ok
            csum = csum + v
        dist_cols = []
        mask_cols = []
        for t in range(_MAX_NEIGHBORS):
            v = sd_cols[t]
            mk = (t < m_sel) & (v <= r_sel * c115) & (v < inf)
            mask_cols.append(jnp.where(mk, jnp.int32(1), jnp.int32(0)))
            dist_cols.append(jnp.where(mk, v, 0.0))
        sd_ref[0] = jnp.concatenate(dist_cols, axis=1)
        gi_ref[0] = jnp.concatenate(gi_cols, axis=1)
        mask_ref[0] = jnp.concatenate(mask_cols, axis=1)


def _topk_candidates(q2, k2, pos, keys_t):
    q2_3 = q2.reshape(_NBLK, _R, 1)
    k2_3 = k2.reshape(1, _NSHIFT, _N).transpose(1, 0, 2)
    pos_3 = pos.reshape(_NBLK, _R, 3)
    kt_3 = keys_t.reshape(3, _NSHIFT, _N).transpose(1, 0, 2)
    dist, gi, mask = pl.pallas_call(
        _select_body,
        grid=(_NBLK, _NSHIFT),
        in_specs=[
            pl.BlockSpec((1, _R, 1), lambda i, s: (i, 0, 0)),
            pl.BlockSpec((1, 1, _N), lambda i, s: (s, 0, 0)),
            pl.BlockSpec((1, _R, 3), lambda i, s: (i, 0, 0)),
            pl.BlockSpec((1, 3, _N), lambda i, s: (s, 0, 0)),
        ],
        out_specs=[
            pl.BlockSpec((1, _R, _MAX_NEIGHBORS), lambda i, s: (i, 0, 0)),
            pl.BlockSpec((1, _R, _MAX_NEIGHBORS), lambda i, s: (i, 0, 0)),
            pl.BlockSpec((1, _R, _MAX_NEIGHBORS), lambda i, s: (i, 0, 0)),
        ],
        out_shape=[
            jax.ShapeDtypeStruct((_NBLK, _R, _MAX_NEIGHBORS), jnp.float32),
            jax.ShapeDtypeStruct((_NBLK, _R, _MAX_NEIGHBORS), jnp.int32),
            jax.ShapeDtypeStruct((_NBLK, _R, _MAX_NEIGHBORS), jnp.int32),
        ],
        scratch_shapes=[
            pltpu.VMEM((_R, _N), jnp.float32),
            pltpu.VMEM((_R, _N), jnp.int32),
        ],
    )(q2_3, k2_3, pos_3, kt_3)
    dist = dist.reshape(_N, _MAX_NEIGHBORS)
    gi = gi.reshape(_N, _MAX_NEIGHBORS)
    mask = mask.reshape(_N, _MAX_NEIGHBORS)
    return dist, gi, mask


_NEDGE = _N * _MAX_NEIGHBORS
_KPAD = 8
_NW = 32
_EPW = _NEDGE // _NW
_L = 16


def _sc_vec_body(gi_hbm, mask_hbm, pos_hbm, off_hbm, out_hbm,
                 idx_v, mask_v, pos_v, off_v, out_v):
    wid = lax.axis_index("s") * 2 + lax.axis_index("c")
    pltpu.sync_copy(gi_hbm.at[wid], idx_v)
    pltpu.sync_copy(mask_hbm.at[wid], mask_v)
    pltpu.sync_copy(pos_hbm, pos_v)
    pltpu.sync_copy(off_hbm, off_v)

    lane = lax.iota(jnp.int32, _L)
    zero = jnp.zeros((_L,), jnp.float32)
    base = wid * _EPW
    for t in range(_EPW // _L):
        e = t * _L + lane
        g = idx_v[pl.ds(t * _L, _L)]
        j3 = jnp.bitwise_and(g, _N - 1) * 3
        s3 = lax.shift_right_logical(g, 10) * 3
        i3 = (lax.shift_right_logical(base + e, 5)) * 3
        keep = mask_v[pl.ds(t * _L, _L)] != 0
        for c in range(3):
            kc = plsc.load_gather(pos_v, [j3 + c]) + plsc.load_gather(off_v, [s3 + c])
            pc = plsc.load_gather(pos_v, [i3 + c])
            vc = jnp.where(keep, kc - pc, zero)
            plsc.store_scatter(out_v, [e * 3 + c], vc)
    pltpu.sync_copy(out_v, out_hbm.at[wid])


def _sc_vec(gi, mask_i32, pos, offsets):
    gi_2 = gi.reshape(_NW, _EPW)
    mask_2 = mask_i32.reshape(_NW, _EPW)
    pos_flat = pos.reshape(_N * 3)
    off_flat = offsets.reshape(_NSHIFT * 3)
    mesh = plsc.VectorSubcoreMesh(core_axis_name="c", subcore_axis_name="s")
    fn = functools.partial(
        pl.kernel,
        mesh=mesh,
        compiler_params=pltpu.CompilerParams(needs_layout_passes=False),
        out_type=jax.ShapeDtypeStruct((_NW, _EPW * 3), jnp.float32),
        scratch_types=[
            pltpu.VMEM((_EPW,), jnp.int32),
            pltpu.VMEM((_EPW,), jnp.int32),
            pltpu.VMEM((_N * 3,), jnp.float32),
            pltpu.VMEM((_NSHIFT * 3,), jnp.float32),
            pltpu.VMEM((_EPW * 3,), jnp.float32),
        ],
    )(_sc_vec_body)
    out = fn(gi_2, mask_2, pos_flat, off_flat)
    return out.reshape(_N, _MAX_NEIGHBORS, 3)


def kernel(frac_coords, cell):
    n = frac_coords.shape[0]
    pos = frac_coords @ cell
    r = jnp.arange(-1, 2)
    shifts = jnp.stack(jnp.meshgrid(r, r, r, indexing="ij"), axis=-1)
    shifts = shifts.reshape(-1, 3).astype(cell.dtype)
    offsets = shifts @ cell
    keys = (pos[None, :, :] + offsets[:, None, :]).reshape(-1, 3)
    pos, keys = jax.lax.optimization_barrier((pos, keys))
    q2 = jnp.sum(pos * pos, axis=1)
    k2 = jnp.sum(keys * keys, axis=1)

    dist, idx_k, mask_i32 = _topk_candidates(q2, k2, pos, keys.T)
    mask = mask_i32 != 0

    vec = _sc_vec(idx_k, mask_i32, pos, offsets)
    dst = idx_k % n
    src = jnp.broadcast_to(jnp.arange(n)[:, None], dst.shape)
    edge_index = jnp.stack([src.reshape(-1), dst.reshape(-1)], axis=0)
    return edge_index, vec, dist, mask


# speedup vs baseline: 1.7876x; 1.1614x over previous
import functools

import jax
import jax.numpy as jnp
from jax import lax
from jax.experimental import pallas as pl
from jax.experimental.pallas import tpu as pltpu
from jax.experimental.pallas import tpu_sc as plsc

_MAX_NEIGHBORS = 32
_CUTOFF = 10.0
_TOL = 0.15
_N = 1024
_R = 512
_NBLK = _N // _R
_NSHIFT = 27
_K1 = _MAX_NEIGHBORS + 1
_OUTW = 64


def _select_body(q2_ref, k2_ref, pos_ref, kt_ref, sd_ref, gi_ref, mask_ref,
                 dmin2, gidx):
    i = pl.program_id(0)
    s = pl.program_id(1)
    p = pos_ref[0]
    kt = kt_ref[0]
    m = jax.lax.dot_general(
        p, kt, dimension_numbers=(((1,), (0,)), ((), ())),
        preferred_element_type=jnp.float32,
    )
    q2 = q2_ref[0]
    k2 = k2_ref[0]
    d2 = (q2 + k2) - 2.0 * m
    colj = jax.lax.broadcasted_iota(jnp.int32, (_R, _N), 1)

    @pl.when(s == 0)
    def _():
        dmin2[...] = d2
        gidx[...] = colj

    @pl.when((s > 0) & (s != 13))
    def _():
        dm = dmin2[...]
        upd = d2 < dm
        dmin2[...] = jnp.where(upd, d2, dm)
        gidx[...] = jnp.where(upd, s * _N + colj, gidx[...])

    @pl.when(s == 13)
    def _():
        rowg = i * _R + jax.lax.broadcasted_iota(jnp.int32, (_R, _N), 0)
        dm = dmin2[...]
        upd = (d2 < dm) & (colj != rowg)
        dmin2[...] = jnp.where(upd, d2, dm)
        gidx[...] = jnp.where(upd, s * _N + colj, gidx[...])

    @pl.when(s == _NSHIFT - 1)
    def _():
        inf = jnp.float32(jnp.inf)
        dcur = jnp.sqrt(jnp.maximum(dmin2[...], 1e-12))
        dcur = jnp.where(dcur > _CUTOFF, inf, dcur)
        g = gidx[...]
        big_i = jnp.int32(2**30)
        c115 = jnp.float32(1.0 + _TOL)
        csum = jnp.zeros((_R, 1), jnp.float32)
        found = jnp.zeros((_R, 1), jnp.bool_)
        m_sel = jnp.full((_R, 1), _MAX_NEIGHBORS, jnp.int32)
        r_sel = jnp.full((_R, 1), _CUTOFF, jnp.float32)
        sd_cols = []
        gi_cols = []
        for t in range(_K1):
            v = jnp.min(dcur, axis=1, keepdims=True)
            tie = dcur == v
            gm = jnp.min(jnp.where(tie, g, big_i), axis=1, keepdims=True)
            if t < _MAX_NEIGHBORS:
                sd_cols.append(v)
                gi_cols.append(gm)
                dcur = jnp.where(g == gm, inf, dcur)
            if t >= 3:
                r_m = csum / jnp.float32(t - 2)
                ok = r_m < v
                newly = ok & (~found)
                m_sel = jnp.where(newly, t, m_sel)
                r_sel = jnp.where(newly, r_m, r_sel)
                found = found | ok
            csum = csum + v
        dist_cols = []
        mask_cols = []
        for t in range(_MAX_NEIGHBORS):
            v = sd_cols[t]
            mk = (t < m_sel) & (v <= r_sel * c115) & (v < inf)
            mask_cols.append(jnp.where(mk, jnp.int32(1), jnp.int32(0)))
            dist_cols.append(jnp.where(mk, v, 0.0))
        sd_ref[0] = jnp.concatenate(dist_cols, axis=1)
        gi_ref[0] = jnp.concatenate(gi_cols, axis=1)
        mask_ref[0] = jnp.concatenate(mask_cols, axis=1)


def _topk_candidates(q2, k2, pos, keys_t):
    q2_3 = q2.reshape(_NBLK, _R, 1)
    k2_3 = k2.reshape(1, _NSHIFT, _N).transpose(1, 0, 2)
    pos_3 = pos.reshape(_NBLK, _R, 3)
    kt_3 = keys_t.reshape(3, _NSHIFT, _N).transpose(1, 0, 2)
    dist, gi, mask = pl.pallas_call(
        _select_body,
        grid=(_NBLK, _NSHIFT),
        in_specs=[
            pl.BlockSpec((1, _R, 1), lambda i, s: (i, 0, 0)),
            pl.BlockSpec((1, 1, _N), lambda i, s: (s, 0, 0)),
            pl.BlockSpec((1, _R, 3), lambda i, s: (i, 0, 0)),
            pl.BlockSpec((1, 3, _N), lambda i, s: (s, 0, 0)),
        ],
        out_specs=[
            pl.BlockSpec((1, _R, _MAX_NEIGHBORS), lambda i, s: (i, 0, 0)),
            pl.BlockSpec((1, _R, _MAX_NEIGHBORS), lambda i, s: (i, 0, 0)),
            pl.BlockSpec((1, _R, _MAX_NEIGHBORS), lambda i, s: (i, 0, 0)),
        ],
        out_shape=[
            jax.ShapeDtypeStruct((_NBLK, _R, _MAX_NEIGHBORS), jnp.float32),
            jax.ShapeDtypeStruct((_NBLK, _R, _MAX_NEIGHBORS), jnp.int32),
            jax.ShapeDtypeStruct((_NBLK, _R, _MAX_NEIGHBORS), jnp.int32),
        ],
        scratch_shapes=[
            pltpu.VMEM((_R, _N), jnp.float32),
            pltpu.VMEM((_R, _N), jnp.int32),
        ],
    )(q2_3, k2_3, pos_3, kt_3)
    dist = dist.reshape(_N, _MAX_NEIGHBORS)
    gi = gi.reshape(_N, _MAX_NEIGHBORS)
    mask = mask.reshape(_N, _MAX_NEIGHBORS)
    return dist, gi, mask


_NEDGE = _N * _MAX_NEIGHBORS
_KPAD = 8
_NW = 32
_EPW = _NEDGE // _NW
_L = 16


def _sc_vec_body(gi_hbm, mask_hbm, pos_hbm, off_hbm, out_hbm,
                 idx_v, mask_v, pos_v, off_v, out_v):
    wid = lax.axis_index("s") * 2 + lax.axis_index("c")
    pltpu.sync_copy(gi_hbm.at[wid], idx_v)
    pltpu.sync_copy(mask_hbm.at[wid], mask_v)
    pltpu.sync_copy(pos_hbm, pos_v)
    pltpu.sync_copy(off_hbm, off_v)

    lane = lax.iota(jnp.int32, _L)
    zero = jnp.zeros((_L,), jnp.float32)
    base = wid * _EPW
    for t in range(_EPW // _L):
        e = t * _L + lane
        g = idx_v[pl.ds(t * _L, _L)]
        j3 = jnp.bitwise_and(g, _N - 1) * 3
        s3 = lax.shift_right_logical(g, 10) * 3
        i3 = (lax.shift_right_logical(base + e, 5)) * 3
        keep = mask_v[pl.ds(t * _L, _L)] != 0
        for c in range(3):
            kc = plsc.load_gather(pos_v, [j3 + c]) + plsc.load_gather(off_v, [s3 + c])
            pc = plsc.load_gather(pos_v, [i3 + c])
            vc = jnp.where(keep, kc - pc, zero)
            plsc.store_scatter(out_v, [e * 3 + c], vc)
    pltpu.sync_copy(out_v, out_hbm.at[wid])


def _sc_vec(gi, mask_i32, pos, offsets):
    gi_2 = gi.reshape(_NW, _EPW)
    mask_2 = mask_i32.reshape(_NW, _EPW)
    pos_flat = pos.reshape(_N * 3)
    off_flat = offsets.reshape(_NSHIFT * 3)
    mesh = plsc.VectorSubcoreMesh(core_axis_name="c", subcore_axis_name="s")
    fn = functools.partial(
        pl.kernel,
        mesh=mesh,
        compiler_params=pltpu.CompilerParams(needs_layout_passes=False),
        out_type=jax.ShapeDtypeStruct((_NW, _EPW * 3), jnp.float32),
        scratch_types=[
            pltpu.VMEM((_EPW,), jnp.int32),
            pltpu.VMEM((_EPW,), jnp.int32),
            pltpu.VMEM((_N * 3,), jnp.float32),
            pltpu.VMEM((_NSHIFT * 3,), jnp.float32),
            pltpu.VMEM((_EPW * 3,), jnp.float32),
        ],
    )(_sc_vec_body)
    out = fn(gi_2, mask_2, pos_flat, off_flat)
    return out.reshape(_N, _MAX_NEIGHBORS, 3)


def kernel(frac_coords, cell):
    n = frac_coords.shape[0]
    pos = frac_coords @ cell
    r = jnp.arange(-1, 2)
    shifts = jnp.stack(jnp.meshgrid(r, r, r, indexing="ij"), axis=-1)
    shifts = shifts.reshape(-1, 3).astype(cell.dtype)
    offsets = shifts @ cell
    keys = (pos[None, :, :] + offsets[:, None, :]).reshape(-1, 3)
    pos, keys = jax.lax.optimization_barrier((pos, keys))
    q2 = jnp.sum(pos * pos, axis=1)
    k2 = jnp.sum(keys * keys, axis=1)

    dist, idx_k, mask_i32 = _topk_candidates(q2, k2, pos, keys.T)
    mask = mask_i32 != 0

    vec = _sc_vec(idx_k, mask_i32, pos, offsets)
    dst = idx_k % n
    src = jnp.broadcast_to(jnp.arange(n)[:, None], dst.shape)
    edge_index = jnp.stack([src.reshape(-1), dst.reshape(-1)], axis=0)
    return edge_index, vec, dist, mask
